# baseline (speedup 1.0000x reference)
import functools

import jax
import jax.numpy as jnp
from jax import lax
from jax.experimental import pallas as pl
from jax.experimental.pallas import tpu as pltpu
from jax.experimental.pallas import tpu_sc as plsc

NC = 2
NS = 16
NW = NC * NS
L = 16

D_MODEL = 128
MONTH_SIZE = 13
DAY_SIZE = 32
COMB = MONTH_SIZE * DAY_SIZE

BATCH = 4096
SEQ = 200
N_ROWS = BATCH * SEQ

SC_ROWS = 98304
ROWS_PER_TILE = SC_ROWS // NW
GROUP = 128
NGROUPS = ROWS_PER_TILE // GROUP
NBUF = 4
NCHUNK = 4
CHUNK_ROWS = ROWS_PER_TILE // NCHUNK

TC_BLK = 1024
TC_ROWS = N_ROWS - SC_ROWS


def _sc_body(tf_hbm, month_hbm, day_hbm, out_hbm,
             month_v, day_v, chunk_v, comb_sp, tf_v, idx_v,
             rows0, rows1, rows2, rows3,
             sg0, sg1, sg2, sg3, sw0, sw1, sw2, sw3):
    rows_v = (rows0, rows1, rows2, rows3)
    sem_g = (sg0, sg1, sg2, sg3)
    sem_w = (sw0, sw1, sw2, sw3)
    cid = lax.axis_index("c")
    sid = lax.axis_index("s")
    wid = sid * NC + cid
    base = wid * ROWS_PER_TILE

    @pl.when(sid == 0)
    def _build():
        pltpu.sync_copy(month_hbm, month_v)
        pltpu.sync_copy(day_hbm, day_v)

        def mloop(m, carry):
            for ch in range(D_MODEL // L):
                sl = pl.ds(ch * L, L)
                mv = month_v[m, sl]
                for dd in range(DAY_SIZE):
                    chunk_v[dd, sl] = mv + day_v[dd, sl]
            pltpu.sync_copy(chunk_v, comb_sp.at[pl.ds(m * DAY_SIZE, DAY_SIZE)])
            return carry

        lax.fori_loop(0, MONTH_SIZE, mloop, 0)

    plsc.subcore_barrier()

    lanes = lax.iota(jnp.int32, L)

    def chunk_pass(c, carry):
        pltpu.sync_copy(
            tf_hbm.at[pl.ds((base + c * CHUNK_ROWS) * 3, CHUNK_ROWS * 3)],
            tf_v)

        def dloop(k, carry2):
            pos = lanes * 3 + k * (L * 3)
            m = plsc.load_gather(tf_v, [pos])
            d = plsc.load_gather(tf_v, [pos + 1])
            idx_v[pl.ds(c * CHUNK_ROWS + k * L, L)] = m * DAY_SIZE + d
            return carry2

        return lax.fori_loop(0, CHUNK_ROWS // L, dloop, carry)

    lax.fori_loop(0, NCHUNK, chunk_pass, 0)

    def g_copy(j, b):
        return pltpu.make_async_copy(
            comb_sp.at[idx_v.at[pl.ds(j * GROUP, GROUP)]], rows_v[b],
            sem_g[b])

    def w_copy(j, b):
        return pltpu.make_async_copy(
            rows_v[b], out_hbm.at[pl.ds(base + j * GROUP, GROUP)], sem_w[b])

    LOOKAHEAD = NBUF // 2
    for b in range(LOOKAHEAD):
        g_copy(b, b).start()

    def gloop(jj, carry):
        for b in range(NBUF):
            j = jj * NBUF + b
            g_copy(j, b).wait()
            w_copy(j, b).start()
            bn = (b + LOOKAHEAD) % NBUF

            @pl.when(j + LOOKAHEAD <= NGROUPS - 1)
            def _refill():
                @pl.when(j >= LOOKAHEAD)
                def _drain():
                    w_copy(j - LOOKAHEAD, bn).wait()

                g_copy(j + LOOKAHEAD, bn).start()
        return carry

    lax.fori_loop(0, NGROUPS // NBUF, gloop, 0)
    for b in range(NBUF):
        jt = NGROUPS - NBUF + b
        w_copy(jt, jt % NBUF).wait()


@functools.partial(
    pl.kernel,
    out_type=jax.ShapeDtypeStruct((N_ROWS, D_MODEL), jnp.float32),
    mesh=plsc.VectorSubcoreMesh(core_axis_name="c", subcore_axis_name="s"),
    compiler_params=pltpu.CompilerParams(needs_layout_passes=False),
    scratch_types=[
        pltpu.VMEM((MONTH_SIZE, D_MODEL), jnp.float32),
        pltpu.VMEM((DAY_SIZE, D_MODEL), jnp.float32),
        pltpu.VMEM((DAY_SIZE, D_MODEL), jnp.float32),
        pltpu.VMEM_SHARED((COMB, D_MODEL), jnp.float32),
        pltpu.VMEM((CHUNK_ROWS * 3,), jnp.int32),
        pltpu.VMEM((ROWS_PER_TILE,), jnp.int32),
        pltpu.VMEM((GROUP, D_MODEL), jnp.float32),
        pltpu.VMEM((GROUP, D_MODEL), jnp.float32),
        pltpu.VMEM((GROUP, D_MODEL), jnp.float32),
        pltpu.VMEM((GROUP, D_MODEL), jnp.float32),
        pltpu.SemaphoreType.DMA,
        pltpu.SemaphoreType.DMA,
        pltpu.SemaphoreType.DMA,
        pltpu.SemaphoreType.DMA,
        pltpu.SemaphoreType.DMA,
        pltpu.SemaphoreType.DMA,
        pltpu.SemaphoreType.DMA,
        pltpu.SemaphoreType.DMA,
    ],
)
def _sc_embed(tf_hbm, month_hbm, day_hbm, out_hbm, *scratch):
    _sc_body(tf_hbm, month_hbm, day_hbm, out_hbm, *scratch)


def _tc_kernel(tf_ref, month_ref, day_ref, sc_ref, out_ref):
    m = tf_ref[0, :]
    d = tf_ref[1, :]
    iota_m = lax.broadcasted_iota(jnp.int32, (TC_BLK, MONTH_SIZE), 1)
    iota_d = lax.broadcasted_iota(jnp.int32, (TC_BLK, DAY_SIZE), 1)
    oh_m = (m[:, None] == iota_m).astype(jnp.float32)
    oh_d = (d[:, None] == iota_d).astype(jnp.float32)
    out_ref[...] = (
        jnp.dot(oh_m, month_ref[...], preferred_element_type=jnp.float32)
        + jnp.dot(oh_d, day_ref[...], preferred_element_type=jnp.float32))


_tc_embed = pl.pallas_call(
    _tc_kernel,
    grid=(TC_ROWS // TC_BLK,),
    in_specs=[
        pl.BlockSpec((3, TC_BLK), lambda i: (0, SC_ROWS // TC_BLK + i)),
        pl.BlockSpec((MONTH_SIZE, D_MODEL), lambda i: (0, 0)),
        pl.BlockSpec((DAY_SIZE, D_MODEL), lambda i: (0, 0)),
        pl.BlockSpec(memory_space=pl.ANY),
    ],
    out_specs=pl.BlockSpec((TC_BLK, D_MODEL),
                           lambda i: (SC_ROWS // TC_BLK + i, 0)),
    out_shape=jax.ShapeDtypeStruct((N_ROWS, D_MODEL), jnp.float32),
    input_output_aliases={3: 0},
)


def kernel(time_features, month_table, day_table, weekday_table):
    tf = time_features.astype(jnp.int32)
    sc_out = _sc_embed(tf.reshape(-1), month_table, day_table)
    tf_t = tf.reshape(N_ROWS, 3).T
    out = _tc_embed(tf_t, month_table, day_table, sc_out)
    return out.reshape(BATCH, SEQ, D_MODEL)
